# baseline (speedup 1.0000x reference)
import functools

import jax
import jax.numpy as jnp
from jax import lax
from jax.experimental import pallas as pl
from jax.experimental.pallas import tpu as pltpu
from jax.experimental.pallas import tpu_sc as plsc

LANES = 16
NF = 16
NBUF = 16
SPS = 1
GRP = NBUF * SPS


def _sc_dims():
    try:
        info = plsc.get_sparse_core_info()
        return info.num_cores, info.num_subcores
    except Exception:
        return 2, 16


def _make_body(nc, bpw, vocab):
    ngrp = bpw // GRP

    def body(users_hbm, items_hbm, uft_hbm, ift_hbm, out_hbm,
             idx_u, idx_v, u_bufs, v_bufs, out_v, *sems):
        wid = lax.axis_index("s") * nc + lax.axis_index("c")
        base = wid * bpw

        cp_u = pltpu.async_copy(users_hbm.at[wid], idx_u, sems[0])
        cp_v = pltpu.async_copy(items_hbm.at[wid], idx_v, sems[1])
        cp_u.wait()
        cp_v.wait()

        lane = lax.iota(jnp.int32, LANES)
        perms = [lane ^ d for d in (1, 2, 4, 8)]

        def block_base(i):
            return pl.multiple_of((i // 128) * 128, 128)

        def fire_batch(s, iv_u, iv_v, t0):
            for t in range(SPS):
                iu = iv_u[t0 + t]
                ivv = iv_v[t0 + t]
                pltpu.async_copy(
                    uft_hbm.at[:, pl.ds(block_base(iu), 128)],
                    u_bufs.at[s * SPS + t], sems[s])
                pltpu.async_copy(
                    ift_hbm.at[:, pl.ds(block_base(ivv), 128)],
                    v_bufs.at[s * SPS + t], sems[s])

        def drain_batch(s):
            for t in range(SPS):
                pltpu.make_async_copy(
                    uft_hbm.at[:, pl.ds(0, 128)], u_bufs.at[s * SPS + t],
                    sems[s]).wait()
                pltpu.make_async_copy(
                    ift_hbm.at[:, pl.ds(0, 128)], v_bufs.at[s * SPS + t],
                    sems[s]).wait()

        iv_u0 = idx_u[pl.ds(0, GRP)]
        iv_v0 = idx_v[pl.ds(0, GRP)]
        for s in range(NBUF):
            fire_batch(s, iv_u0, iv_v0, s * SPS)

        def grp(g, _):
            goff = g * GRP
            iv_u = idx_u[pl.ds(goff, GRP)]
            iv_v = idx_v[pl.ds(goff, GRP)]
            nof = jnp.minimum(g + 1, ngrp - 1) * GRP
            nu = idx_u[pl.ds(nof, GRP)]
            nv = idx_v[pl.ds(nof, GRP)]
            acc = jnp.zeros((LANES,), jnp.float32)
            for s in range(NBUF):
                drain_batch(s)
                for t in range(SPS):
                    j = s * SPS + t
                    iu = iv_u[j]
                    ivv = iv_v[j]
                    lu = jnp.broadcast_to(iu - block_base(iu), (LANES,))
                    lv = jnp.broadcast_to(ivv - block_base(ivv), (LANES,))
                    gu = plsc.load_gather(u_bufs.at[s * SPS + t], [lane, lu])
                    gv = plsc.load_gather(v_bufs.at[s * SPS + t], [lane, lv])
                    p = gu * gv
                    for perm in perms:
                        p = p + p.at[perm].get(mode="promise_in_bounds")
                    acc = jnp.where(lane == j, p, acc)

                @pl.when(g + 1 < ngrp)
                def _refire():
                    fire_batch(s, nu, nv, s * SPS)

            out_v[pl.ds(goff, GRP)] = acc
            return 0

        lax.fori_loop(0, ngrp, grp, 0)
        pltpu.sync_copy(out_v, out_hbm.at[pl.ds(base, bpw)])

    return body


@jax.jit
def kernel(x, user_factors, item_factors):
    nc, ns = _sc_dims()
    nw = nc * ns
    batch = x.shape[0]
    vocab = user_factors.shape[0]
    assert batch % (nw * GRP) == 0
    bpw = batch // nw

    users = x[:, 0].astype(jnp.int32).reshape(nw, bpw)
    items = x[:, 1].astype(jnp.int32).reshape(nw, bpw)

    mesh = plsc.VectorSubcoreMesh(core_axis_name="c", subcore_axis_name="s")
    fn = pl.kernel(
        _make_body(nc, bpw, vocab),
        out_type=jax.ShapeDtypeStruct((batch,), jnp.float32),
        mesh=mesh,
        scratch_types=[
            pltpu.VMEM((bpw,), jnp.int32),
            pltpu.VMEM((bpw,), jnp.int32),
            pltpu.VMEM((NBUF * SPS, NF, 128), jnp.float32),
            pltpu.VMEM((NBUF * SPS, NF, 128), jnp.float32),
            pltpu.VMEM((bpw,), jnp.float32),
        ] + [pltpu.SemaphoreType.DMA] * NBUF,
        compiler_params=pltpu.CompilerParams(
            disable_bounds_checks=True, needs_layout_passes=False),
    )
    return fn(users, items, user_factors.T, item_factors.T)


# speedup vs baseline: 1.0762x; 1.0762x over previous
import functools

import jax
import jax.numpy as jnp
from jax import lax
from jax.experimental import pallas as pl
from jax.experimental.pallas import tpu as pltpu
from jax.experimental.pallas import tpu_sc as plsc

LANES = 16
NF = 16
NBUF = 8
SPS = 2
GRP = NBUF * SPS


def _sc_dims():
    try:
        info = plsc.get_sparse_core_info()
        return info.num_cores, info.num_subcores
    except Exception:
        return 2, 16


def _make_body(nc, bpw, vocab):
    ngrp = bpw // GRP

    def body(users_hbm, items_hbm, uft_hbm, ift_hbm, out_hbm,
             idx_u, idx_v, u_bufs, v_bufs, out_v, *sems):
        wid = lax.axis_index("s") * nc + lax.axis_index("c")
        base = wid * bpw

        cp_u = pltpu.async_copy(users_hbm.at[wid], idx_u, sems[0])
        cp_v = pltpu.async_copy(items_hbm.at[wid], idx_v, sems[1])
        cp_u.wait()
        cp_v.wait()

        lane = lax.iota(jnp.int32, LANES)
        perms = [lane ^ d for d in (1, 2, 4, 8)]

        def block_base(i):
            return pl.multiple_of((i // 128) * 128, 128)

        def fire_batch(s, iv_u, iv_v, t0):
            for t in range(SPS):
                iu = iv_u[t0 + t]
                ivv = iv_v[t0 + t]
                pltpu.async_copy(
                    uft_hbm.at[:, pl.ds(block_base(iu), 128)],
                    u_bufs.at[s].at[:, pl.ds(t * 128, 128)], sems[s])
                pltpu.async_copy(
                    ift_hbm.at[:, pl.ds(block_base(ivv), 128)],
                    v_bufs.at[s].at[:, pl.ds(t * 128, 128)], sems[s])

        def drain_batch(s):
            pltpu.make_async_copy(
                uft_hbm.at[:, pl.ds(0, SPS * 128)],
                u_bufs.at[s], sems[s]).wait()
            pltpu.make_async_copy(
                ift_hbm.at[:, pl.ds(0, SPS * 128)],
                v_bufs.at[s], sems[s]).wait()

        iv_u0 = idx_u[pl.ds(0, GRP)]
        iv_v0 = idx_v[pl.ds(0, GRP)]
        for s in range(NBUF):
            fire_batch(s, iv_u0, iv_v0, s * SPS)

        def grp(g, _):
            goff = g * GRP
            iv_u = idx_u[pl.ds(goff, GRP)]
            iv_v = idx_v[pl.ds(goff, GRP)]
            nof = jnp.minimum(g + 1, ngrp - 1) * GRP
            nu = idx_u[pl.ds(nof, GRP)]
            nv = idx_v[pl.ds(nof, GRP)]
            acc = jnp.zeros((LANES,), jnp.float32)
            for s in range(NBUF):
                drain_batch(s)
                for t in range(SPS):
                    j = s * SPS + t
                    iu = iv_u[j]
                    ivv = iv_v[j]
                    lu = jnp.broadcast_to(iu - block_base(iu), (LANES,))
                    lv = jnp.broadcast_to(ivv - block_base(ivv), (LANES,))
                    gu = plsc.load_gather(u_bufs.at[s], [lane, lu + t * 128])
                    gv = plsc.load_gather(v_bufs.at[s], [lane, lv + t * 128])
                    p = gu * gv
                    for perm in perms:
                        p = p + p.at[perm].get(mode="promise_in_bounds")
                    acc = jnp.where(lane == j, p, acc)

                @pl.when(g + 1 < ngrp)
                def _refire():
                    fire_batch(s, nu, nv, s * SPS)

            out_v[pl.ds(goff, GRP)] = acc
            return 0

        lax.fori_loop(0, ngrp, grp, 0)
        pltpu.sync_copy(out_v, out_hbm.at[pl.ds(base, bpw)])

    return body


@jax.jit
def kernel(x, user_factors, item_factors):
    nc, ns = _sc_dims()
    nw = nc * ns
    batch = x.shape[0]
    vocab = user_factors.shape[0]
    assert batch % (nw * GRP) == 0
    bpw = batch // nw

    users = x[:, 0].astype(jnp.int32).reshape(nw, bpw)
    items = x[:, 1].astype(jnp.int32).reshape(nw, bpw)

    mesh = plsc.VectorSubcoreMesh(core_axis_name="c", subcore_axis_name="s")
    fn = pl.kernel(
        _make_body(nc, bpw, vocab),
        out_type=jax.ShapeDtypeStruct((batch,), jnp.float32),
        mesh=mesh,
        scratch_types=[
            pltpu.VMEM((bpw,), jnp.int32),
            pltpu.VMEM((bpw,), jnp.int32),
            pltpu.VMEM((NBUF, NF, SPS * 128), jnp.float32),
            pltpu.VMEM((NBUF, NF, SPS * 128), jnp.float32),
            pltpu.VMEM((bpw,), jnp.float32),
        ] + [pltpu.SemaphoreType.DMA] * NBUF,
        compiler_params=pltpu.CompilerParams(
            disable_bounds_checks=True, needs_layout_passes=False),
    )
    return fn(users, items, user_factors.T, item_factors.T)
